# baseline (speedup 1.0000x reference)
import functools

import jax
import jax.numpy as jnp
from jax import lax
from jax.experimental import pallas as pl
from jax.experimental.pallas import tpu as pltpu
from jax.experimental.pallas import tpu_sc as plsc

_T = 2048
_D = 1024
_INTER = 2048
_H = 4
_KP = 8
_NK = 128
_SEL = _H * _KP

_BT = 256
_NC, _NS = 2, 16
_NW = _NC * _NS
_TPW = _T // _NW


def _top8f(s, payload=None):
    r, n = s.shape
    iota = lax.broadcasted_iota(jnp.int32, s.shape, 1).astype(jnp.float32)
    big = jnp.float32(n)
    ss, ii, pp = [], [], []
    cur = s
    for _ in range(8):
        m = jnp.max(cur, axis=1, keepdims=True)
        ismax = cur == m
        first = jnp.min(jnp.where(ismax, iota, big), axis=1, keepdims=True)
        hit = iota == first
        ss.append(m)
        ii.append(first)
        if payload is not None:
            pp.append(jnp.sum(jnp.where(hit, payload, 0.0), axis=1,
                              keepdims=True))
        cur = jnp.where(hit, -jnp.inf, cur)
    out_s = jnp.concatenate(ss, axis=1)
    out_i = jnp.concatenate(ii, axis=1)
    if payload is not None:
        return out_s, out_i, jnp.concatenate(pp, axis=1)
    return out_s, out_i


def _routing_body(h_ref, wq_ref, k2_ref, idx_ref, prob_ref):
    h = h_ref[...]
    q = jnp.dot(h, wq_ref[...], preferred_element_type=jnp.float32)
    sims = []
    for p in range(2):
        for hh in range(_H):
            c0 = p * (_H * 32) + hh * 32
            qph = q[:, c0:c0 + 32]
            r0 = (p * _H + hh) * 32
            kph = k2_ref[r0:r0 + 32, :]
            sims.append(jnp.dot(qph, kph,
                                preferred_element_type=jnp.float32))
    s8a, i8a = _top8f(jnp.concatenate(sims, axis=0))

    all_s_parts, all_i_parts = [], []
    for hh in range(_H):
        sx = s8a[hh * _BT:(hh + 1) * _BT]
        ix = i8a[hh * _BT:(hh + 1) * _BT]
        sy = s8a[(_H + hh) * _BT:(_H + hh + 1) * _BT]
        iy = i8a[(_H + hh) * _BT:(_H + hh + 1) * _BT]
        all_s_parts.append(
            jnp.concatenate([sx[:, a:a + 1] + sy for a in range(8)], axis=1))
        all_i_parts.append(
            jnp.concatenate([ix[:, a:a + 1] * float(_NK) + iy
                             for a in range(8)], axis=1))
    s8, _, e8 = _top8f(jnp.concatenate(all_s_parts, axis=0),
                       payload=jnp.concatenate(all_i_parts, axis=0))
    m = jnp.max(s8, axis=1, keepdims=True)
    e = jnp.exp(s8 - m)
    sm = e / jnp.sum(e, axis=1, keepdims=True)
    e8i = e8.astype(jnp.int32)
    idx_ref[...] = jnp.concatenate(
        [e8i[hh * _BT:(hh + 1) * _BT] for hh in range(_H)], axis=1)
    prob_ref[...] = jnp.concatenate(
        [sm[hh * _BT:(hh + 1) * _BT] for hh in range(_H)], axis=1)


def _routing(h2, wq, k2, interpret=False):
    return pl.pallas_call(
        _routing_body,
        grid=(_T // _BT,),
        in_specs=[
            pl.BlockSpec((_BT, _D), lambda i: (i, 0)),
            pl.BlockSpec(wq.shape, lambda i: (0, 0)),
            pl.BlockSpec(k2.shape, lambda i: (0, 0)),
        ],
        out_specs=[
            pl.BlockSpec((_BT, _SEL), lambda i: (i, 0)),
            pl.BlockSpec((_BT, _SEL), lambda i: (i, 0)),
        ],
        out_shape=[
            jax.ShapeDtypeStruct((_T, _SEL), jnp.int32),
            jax.ShapeDtypeStruct((_T, _SEL), jnp.float32),
        ],
        interpret=interpret,
    )(h2, wq, k2)


def _mlp_body(h_ref, wg_ref, wu_ref, wd_ref, mlp_ref):
    h = h_ref[...].astype(jnp.bfloat16)
    g = jnp.dot(h, wg_ref[...], preferred_element_type=jnp.float32)
    u = jnp.dot(h, wu_ref[...], preferred_element_type=jnp.float32)
    a = ((g / (1.0 + jnp.exp(-g))) * u).astype(jnp.bfloat16)
    mlp_ref[...] = jnp.dot(a, wd_ref[...], preferred_element_type=jnp.float32)


def _mlp(h2, wg, wu, wd, interpret=False):
    return pl.pallas_call(
        _mlp_body,
        grid=(_T // _BT,),
        in_specs=[
            pl.BlockSpec((_BT, _D), lambda i: (i, 0)),
            pl.BlockSpec(wg.shape, lambda i: (0, 0)),
            pl.BlockSpec(wu.shape, lambda i: (0, 0)),
            pl.BlockSpec(wd.shape, lambda i: (0, 0)),
        ],
        out_specs=pl.BlockSpec((_BT, _D), lambda i: (i, 0)),
        out_shape=jax.ShapeDtypeStruct((_T, _D), jnp.float32),
        interpret=interpret,
    )(h2, wg, wu, wd)


def _add_body(a_ref, b_ref, o_ref):
    o_ref[...] = a_ref[...] + b_ref[...]


def _add(a, b, interpret=False):
    return pl.pallas_call(
        _add_body,
        grid=(_T // _BT,),
        in_specs=[
            pl.BlockSpec((_BT, _D), lambda i: (i, 0)),
            pl.BlockSpec((_BT, _D), lambda i: (i, 0)),
        ],
        out_specs=pl.BlockSpec((_BT, _D), lambda i: (i, 0)),
        out_shape=jax.ShapeDtypeStruct((_T, _D), jnp.float32),
        interpret=interpret,
    )(a, b)


def _sc_experts_body(h_hbm, idx_hbm, prob_hbm, down_hbm, up_hbm, out_hbm,
                     idx_all, prob_all, h_v, out_v, dn0, dn1, up_v,
                     sem_h0, sem_h1, sem_d0, sem_d1, sem_u):
    wid = lax.axis_index("s") * _NC + lax.axis_index("c")
    t0 = wid * _TPW
    lane = lax.iota(jnp.int32, 16)

    pltpu.sync_copy(idx_hbm.at[pl.ds(t0, _TPW)], idx_all)
    pltpu.sync_copy(prob_hbm.at[pl.ds(t0, _TPW)], prob_all)

    pltpu.async_copy(down_hbm.at[idx_all.at[0]], dn0, sem_d0)
    pltpu.async_copy(h_hbm.at[t0], h_v.at[0], sem_h0)

    dns = (dn0, dn1)
    sds = (sem_d0, sem_d1)
    shs = (sem_h0, sem_h1)

    def process(i, b):
        t = t0 + i
        dn_cur, dn_nxt = dns[b], dns[1 - b]
        sd_cur, sd_nxt = sds[b], sds[1 - b]
        sh_cur, sh_nxt = shs[b], shs[1 - b]

        pltpu.async_copy(up_hbm.at[idx_all.at[i]], up_v, sem_u)

        @pl.when(i + 1 < _TPW)
        def _():
            pltpu.async_copy(down_hbm.at[idx_all.at[i + 1]], dn_nxt, sd_nxt)
            pltpu.async_copy(h_hbm.at[t + 1], h_v.at[1 - b], sh_nxt)

        pltpu.make_async_copy(down_hbm.at[idx_all.at[i]], dn_cur, sd_cur).wait()
        pltpu.make_async_copy(h_hbm.at[t], h_v.at[b], sh_cur).wait()

        def dot_chunk(g, accs):
            even = g * 32 + 2 * lane
            brow = jnp.full((16,), b, jnp.int32)
            h0 = plsc.load_gather(h_v, [brow, even])
            h1 = plsc.load_gather(h_v, [brow, even + 1])
            new = []
            for j in range(_SEL):
                w = plsc.bitcast(dn_cur[j, pl.ds(g * 16, 16)], jnp.bfloat16)
                u0, u1 = plsc.unpack(w, format=plsc.PackFormat.INTERLEAVED)
                new.append(accs[j] + h0 * u0 + h1 * u1)
            return tuple(new)

        zero = jnp.zeros((16,), jnp.float32)
        accs = lax.fori_loop(0, _D // 32, dot_chunk, (zero,) * _SEL)

        ews = []
        for j in range(_SEL):
            dj = jnp.sum(accs[j])
            pv = prob_all[i, pl.ds((j // 16) * 16, 16)]
            pj = jnp.sum(jnp.where(lane == (j % 16), pv, 0.0))
            v = jnp.full((16,), dj)
            sv = v / (1.0 + jnp.exp(-v))
            sj = jnp.sum(sv) * (1.0 / 16.0)
            ews.append(sj * pj)

        pltpu.make_async_copy(up_hbm.at[idx_all.at[i]], up_v, sem_u).wait()

        def up_chunk(g, c):
            w = plsc.bitcast(up_v[0, pl.ds(g * 16, 16)], jnp.bfloat16)
            u0, u1 = plsc.unpack(w, format=plsc.PackFormat.INTERLEAVED)
            o0 = ews[0] * u0
            o1 = ews[0] * u1
            for j in range(1, _SEL):
                w = plsc.bitcast(up_v[j, pl.ds(g * 16, 16)], jnp.bfloat16)
                u0, u1 = plsc.unpack(w, format=plsc.PackFormat.INTERLEAVED)
                o0 = o0 + ews[j] * u0
                o1 = o1 + ews[j] * u1
            even = g * 32 + 2 * lane
            plsc.store_scatter(out_v, [even], o0)
            plsc.store_scatter(out_v, [even + 1], o1)
            return c

        lax.fori_loop(0, _D // 32, up_chunk, 0)
        pltpu.sync_copy(out_v, out_hbm.at[t])

    def pair(k, carry):
        process(2 * k, 0)
        process(2 * k + 1, 1)
        return carry

    lax.fori_loop(0, _TPW // 2, pair, 0)


def _sc_experts(h2, idx, prob, down, up):
    mesh = plsc.VectorSubcoreMesh(core_axis_name="c", subcore_axis_name="s",
                                  num_cores=_NC, num_subcores=_NS)
    run = pl.kernel(
        _sc_experts_body,
        out_type=jax.ShapeDtypeStruct((_T, _D), jnp.float32),
        mesh=mesh,
        scratch_types=[
            pltpu.VMEM((_TPW, _SEL), jnp.int32),
            pltpu.VMEM((_TPW, _SEL), jnp.float32),
            pltpu.VMEM((2, _D), jnp.float32),
            pltpu.VMEM((_D,), jnp.float32),
            pltpu.VMEM((_SEL, _D // 2), jnp.int32),
            pltpu.VMEM((_SEL, _D // 2), jnp.int32),
            pltpu.VMEM((_SEL, _D // 2), jnp.int32),
            pltpu.SemaphoreType.DMA,
            pltpu.SemaphoreType.DMA,
            pltpu.SemaphoreType.DMA,
            pltpu.SemaphoreType.DMA,
            pltpu.SemaphoreType.DMA,
        ],
        compiler_params=pltpu.CompilerParams(needs_layout_passes=False),
    )
    return run(h2, idx, prob, down, up)


def _pack_table(table):
    e, d = table.shape
    bf = table.astype(jnp.bfloat16).reshape(e, d // 2, 2)
    return lax.bitcast_convert_type(bf, jnp.int32)


def kernel(hidden_states, Wq, keys_p, down_embed, up_embed, Wg, Wu, Wd):
    b, t, d = hidden_states.shape
    h2 = hidden_states.reshape(t, d)
    k2 = jnp.transpose(keys_p, (2, 0, 3, 1)).reshape(2 * _H * 32, _NK)
    idx, prob = _routing(h2, Wq, k2)
    experts = _sc_experts(h2, idx, prob, _pack_table(down_embed),
                          _pack_table(up_embed))
    mlp = _mlp(h2, Wg.astype(jnp.bfloat16), Wu.astype(jnp.bfloat16),
               Wd.astype(jnp.bfloat16))
    out = _add(mlp, experts)
    return out.reshape(b, t, d)


# speedup vs baseline: 1.4600x; 1.4600x over previous
import functools

import jax
import jax.numpy as jnp
from jax import lax
from jax.experimental import pallas as pl
from jax.experimental.pallas import tpu as pltpu
from jax.experimental.pallas import tpu_sc as plsc

_T = 2048
_D = 1024
_INTER = 2048
_H = 4
_KP = 8
_NK = 128
_SEL = _H * _KP

_BT = 256
_NC, _NS = 2, 16
_NW = _NC * _NS
_TPW = _T // _NW


def _top8f(s, payload=None):
    r, n = s.shape
    iota = lax.broadcasted_iota(jnp.int32, s.shape, 1).astype(jnp.float32)
    big = jnp.float32(n)
    ss, ii, pp = [], [], []
    cur = s
    for _ in range(8):
        m = jnp.max(cur, axis=1, keepdims=True)
        ismax = cur == m
        first = jnp.min(jnp.where(ismax, iota, big), axis=1, keepdims=True)
        hit = iota == first
        ss.append(m)
        ii.append(first)
        if payload is not None:
            pp.append(jnp.sum(jnp.where(hit, payload, 0.0), axis=1,
                              keepdims=True))
        cur = jnp.where(hit, -jnp.inf, cur)
    out_s = jnp.concatenate(ss, axis=1)
    out_i = jnp.concatenate(ii, axis=1)
    if payload is not None:
        return out_s, out_i, jnp.concatenate(pp, axis=1)
    return out_s, out_i


def _routing_body(h_ref, wq_ref, k2_ref, idx_ref, prob_ref):
    h = h_ref[...]
    q = jnp.dot(h, wq_ref[...], preferred_element_type=jnp.float32)
    sims = []
    for p in range(2):
        for hh in range(_H):
            c0 = p * (_H * 32) + hh * 32
            qph = q[:, c0:c0 + 32]
            r0 = (p * _H + hh) * 32
            kph = k2_ref[r0:r0 + 32, :]
            sims.append(jnp.dot(qph, kph,
                                preferred_element_type=jnp.float32))
    s8a, i8a = _top8f(jnp.concatenate(sims, axis=0))

    all_s_parts, all_i_parts = [], []
    for hh in range(_H):
        sx = s8a[hh * _BT:(hh + 1) * _BT]
        ix = i8a[hh * _BT:(hh + 1) * _BT]
        sy = s8a[(_H + hh) * _BT:(_H + hh + 1) * _BT]
        iy = i8a[(_H + hh) * _BT:(_H + hh + 1) * _BT]
        all_s_parts.append(
            jnp.concatenate([sx[:, a:a + 1] + sy for a in range(8)], axis=1))
        all_i_parts.append(
            jnp.concatenate([ix[:, a:a + 1] * float(_NK) + iy
                             for a in range(8)], axis=1))
    s8, _, e8 = _top8f(jnp.concatenate(all_s_parts, axis=0),
                       payload=jnp.concatenate(all_i_parts, axis=0))
    m = jnp.max(s8, axis=1, keepdims=True)
    e = jnp.exp(s8 - m)
    sm = e / jnp.sum(e, axis=1, keepdims=True)
    e8i = e8.astype(jnp.int32)
    idx_ref[...] = jnp.concatenate(
        [e8i[hh * _BT:(hh + 1) * _BT] for hh in range(_H)], axis=1)
    prob_ref[...] = jnp.concatenate(
        [sm[hh * _BT:(hh + 1) * _BT] for hh in range(_H)], axis=1)


def _routing(h2, wq, k2, interpret=False):
    return pl.pallas_call(
        _routing_body,
        grid=(_T // _BT,),
        in_specs=[
            pl.BlockSpec((_BT, _D), lambda i: (i, 0)),
            pl.BlockSpec(wq.shape, lambda i: (0, 0)),
            pl.BlockSpec(k2.shape, lambda i: (0, 0)),
        ],
        out_specs=[
            pl.BlockSpec((_BT, _SEL), lambda i: (i, 0)),
            pl.BlockSpec((_BT, _SEL), lambda i: (i, 0)),
        ],
        out_shape=[
            jax.ShapeDtypeStruct((_T, _SEL), jnp.int32),
            jax.ShapeDtypeStruct((_T, _SEL), jnp.float32),
        ],
        interpret=interpret,
    )(h2, wq, k2)


def _mlp_body(h_ref, wg_ref, wu_ref, wd_ref, mlp_ref):
    h = h_ref[...].astype(jnp.bfloat16)
    g = jnp.dot(h, wg_ref[...], preferred_element_type=jnp.float32)
    u = jnp.dot(h, wu_ref[...], preferred_element_type=jnp.float32)
    a = ((g / (1.0 + jnp.exp(-g))) * u).astype(jnp.bfloat16)
    mlp_ref[...] = jnp.dot(a, wd_ref[...], preferred_element_type=jnp.float32)


def _mlp(h2, wg, wu, wd, interpret=False):
    return pl.pallas_call(
        _mlp_body,
        grid=(_T // _BT,),
        in_specs=[
            pl.BlockSpec((_BT, _D), lambda i: (i, 0)),
            pl.BlockSpec(wg.shape, lambda i: (0, 0)),
            pl.BlockSpec(wu.shape, lambda i: (0, 0)),
            pl.BlockSpec(wd.shape, lambda i: (0, 0)),
        ],
        out_specs=pl.BlockSpec((_BT, _D), lambda i: (i, 0)),
        out_shape=jax.ShapeDtypeStruct((_T, _D), jnp.float32),
        interpret=interpret,
    )(h2, wg, wu, wd)


def _add_body(a_ref, b_ref, o_ref):
    o_ref[...] = a_ref[...] + b_ref[...]


def _add(a, b, interpret=False):
    return pl.pallas_call(
        _add_body,
        grid=(_T // _BT,),
        in_specs=[
            pl.BlockSpec((_BT, _D), lambda i: (i, 0)),
            pl.BlockSpec((_BT, _D), lambda i: (i, 0)),
        ],
        out_specs=pl.BlockSpec((_BT, _D), lambda i: (i, 0)),
        out_shape=jax.ShapeDtypeStruct((_T, _D), jnp.float32),
        interpret=interpret,
    )(a, b)


def _sc_experts_body(h_hbm, idx_hbm, prob_hbm, down_hbm, up_hbm, out_hbm,
                     idx_all, prob_all, h_v, out_v, dn0, dn1, up_v,
                     sem_h0, sem_h1, sem_d0, sem_d1, sem_u):
    wid = lax.axis_index("s") * _NC + lax.axis_index("c")
    t0 = wid * _TPW
    lane = lax.iota(jnp.int32, 16)

    pltpu.sync_copy(idx_hbm.at[pl.ds(t0, _TPW)], idx_all)
    pltpu.sync_copy(prob_hbm.at[pl.ds(t0, _TPW)], prob_all)

    pltpu.async_copy(down_hbm.at[idx_all.at[0]], dn0, sem_d0)
    pltpu.async_copy(h_hbm.at[t0], h_v.at[0], sem_h0)

    dns = (dn0, dn1)
    sds = (sem_d0, sem_d1)
    shs = (sem_h0, sem_h1)

    def process(i, b):
        t = t0 + i
        dn_cur, dn_nxt = dns[b], dns[1 - b]
        sd_cur, sd_nxt = sds[b], sds[1 - b]
        sh_cur, sh_nxt = shs[b], shs[1 - b]

        pltpu.async_copy(up_hbm.at[idx_all.at[i]], up_v, sem_u)

        @pl.when(i + 1 < _TPW)
        def _():
            pltpu.async_copy(down_hbm.at[idx_all.at[i + 1]], dn_nxt, sd_nxt)
            pltpu.async_copy(h_hbm.at[t + 1], h_v.at[1 - b], sh_nxt)

        pltpu.make_async_copy(down_hbm.at[idx_all.at[i]], dn_cur, sd_cur).wait()
        pltpu.make_async_copy(h_hbm.at[t], h_v.at[b], sh_cur).wait()

        def dot_chunk(g, accs):
            even = g * 32 + 2 * lane
            brow = jnp.full((16,), b, jnp.int32)
            h0 = plsc.load_gather(h_v, [brow, even])
            h1 = plsc.load_gather(h_v, [brow, even + 1])
            new = []
            for j in range(_SEL):
                w = dn_cur[j, pl.ds(g * 32, 32)]
                u0, u1 = plsc.unpack(w, format=plsc.PackFormat.INTERLEAVED)
                new.append(accs[j] + h0 * u0 + h1 * u1)
            return tuple(new)

        zero = jnp.zeros((16,), jnp.float32)
        accs = lax.fori_loop(0, _D // 32, dot_chunk, (zero,) * _SEL)

        ews = []
        for j in range(_SEL):
            dj = jnp.sum(accs[j])
            pv = prob_all[i, pl.ds((j // 16) * 16, 16)]
            pj = jnp.sum(jnp.where(lane == (j % 16), pv, 0.0))
            v = jnp.full((16,), dj)
            sv = v / (1.0 + jnp.exp(-v))
            sj = jnp.sum(sv) * (1.0 / 16.0)
            ews.append(sj * pj)

        pltpu.make_async_copy(up_hbm.at[idx_all.at[i]], up_v, sem_u).wait()

        def up_chunk(g, c):
            w = up_v[0, pl.ds(g * 32, 32)]
            u0, u1 = plsc.unpack(w, format=plsc.PackFormat.INTERLEAVED)
            o0 = ews[0] * u0
            o1 = ews[0] * u1
            for j in range(1, _SEL):
                w = up_v[j, pl.ds(g * 32, 32)]
                u0, u1 = plsc.unpack(w, format=plsc.PackFormat.INTERLEAVED)
                o0 = o0 + ews[j] * u0
                o1 = o1 + ews[j] * u1
            even = g * 32 + 2 * lane
            plsc.store_scatter(out_v, [even], o0)
            plsc.store_scatter(out_v, [even + 1], o1)
            return c

        lax.fori_loop(0, _D // 32, up_chunk, 0)
        pltpu.sync_copy(out_v, out_hbm.at[t])

    def pair(k, carry):
        process(2 * k, 0)
        process(2 * k + 1, 1)
        return carry

    lax.fori_loop(0, _TPW // 2, pair, 0)


def _sc_experts(h2, idx, prob, down, up):
    mesh = plsc.VectorSubcoreMesh(core_axis_name="c", subcore_axis_name="s",
                                  num_cores=_NC, num_subcores=_NS)
    run = pl.kernel(
        _sc_experts_body,
        out_type=jax.ShapeDtypeStruct((_T, _D), jnp.float32),
        mesh=mesh,
        scratch_types=[
            pltpu.VMEM((_TPW, _SEL), jnp.int32),
            pltpu.VMEM((_TPW, _SEL), jnp.float32),
            pltpu.VMEM((2, _D), jnp.float32),
            pltpu.VMEM((_D,), jnp.float32),
            pltpu.VMEM((_SEL, _D), jnp.bfloat16),
            pltpu.VMEM((_SEL, _D), jnp.bfloat16),
            pltpu.VMEM((_SEL, _D), jnp.bfloat16),
            pltpu.SemaphoreType.DMA,
            pltpu.SemaphoreType.DMA,
            pltpu.SemaphoreType.DMA,
            pltpu.SemaphoreType.DMA,
            pltpu.SemaphoreType.DMA,
        ],
        compiler_params=pltpu.CompilerParams(needs_layout_passes=False,
                                             use_tc_tiling_on_sc=False),
    )
    return run(h2, idx, prob, down, up)


def kernel(hidden_states, Wq, keys_p, down_embed, up_embed, Wg, Wu, Wd):
    b, t, d = hidden_states.shape
    h2 = hidden_states.reshape(t, d)
    k2 = jnp.transpose(keys_p, (2, 0, 3, 1)).reshape(2 * _H * 32, _NK)
    idx, prob = _routing(h2, Wq, k2)
    experts = _sc_experts(h2, idx, prob,
                          down_embed.astype(jnp.bfloat16),
                          up_embed.astype(jnp.bfloat16))
    mlp = _mlp(h2, Wg.astype(jnp.bfloat16), Wu.astype(jnp.bfloat16),
               Wd.astype(jnp.bfloat16))
    out = _add(mlp, experts)
    return out.reshape(b, t, d)


# speedup vs baseline: 2.0003x; 1.3700x over previous
import functools

import jax
import jax.numpy as jnp
from jax import lax
from jax.experimental import pallas as pl
from jax.experimental.pallas import tpu as pltpu
from jax.experimental.pallas import tpu_sc as plsc

_T = 2048
_D = 1024
_INTER = 2048
_H = 4
_KP = 8
_NK = 128
_SEL = _H * _KP

_BT = 256
_NC, _NS = 2, 16
_NW = _NC * _NS
_TPW = _T // _NW


def _top8f(s, payload=None):
    r, n = s.shape
    iota = lax.broadcasted_iota(jnp.int32, s.shape, 1).astype(jnp.float32)
    big = jnp.float32(n)
    ss, ii, pp = [], [], []
    cur = s
    for _ in range(8):
        m = jnp.max(cur, axis=1, keepdims=True)
        ismax = cur == m
        first = jnp.min(jnp.where(ismax, iota, big), axis=1, keepdims=True)
        hit = iota == first
        ss.append(m)
        ii.append(first)
        if payload is not None:
            pp.append(jnp.sum(jnp.where(hit, payload, 0.0), axis=1,
                              keepdims=True))
        cur = jnp.where(hit, -jnp.inf, cur)
    out_s = jnp.concatenate(ss, axis=1)
    out_i = jnp.concatenate(ii, axis=1)
    if payload is not None:
        return out_s, out_i, jnp.concatenate(pp, axis=1)
    return out_s, out_i


def _routing_body(h_ref, wq_ref, k2_ref, idx_ref, prob_ref):
    h = h_ref[...]
    q = jnp.dot(h, wq_ref[...], preferred_element_type=jnp.float32)
    sims = []
    for p in range(2):
        for hh in range(_H):
            c0 = p * (_H * 32) + hh * 32
            qph = q[:, c0:c0 + 32]
            r0 = (p * _H + hh) * 32
            kph = k2_ref[r0:r0 + 32, :]
            sims.append(jnp.dot(qph, kph,
                                preferred_element_type=jnp.float32))
    s8a, i8a = _top8f(jnp.concatenate(sims, axis=0))

    all_s_parts, all_i_parts = [], []
    for hh in range(_H):
        sx = s8a[hh * _BT:(hh + 1) * _BT]
        ix = i8a[hh * _BT:(hh + 1) * _BT]
        sy = s8a[(_H + hh) * _BT:(_H + hh + 1) * _BT]
        iy = i8a[(_H + hh) * _BT:(_H + hh + 1) * _BT]
        all_s_parts.append(
            jnp.concatenate([sx[:, a:a + 1] + sy for a in range(8)], axis=1))
        all_i_parts.append(
            jnp.concatenate([ix[:, a:a + 1] * float(_NK) + iy
                             for a in range(8)], axis=1))
    s8, _, e8 = _top8f(jnp.concatenate(all_s_parts, axis=0),
                       payload=jnp.concatenate(all_i_parts, axis=0))
    m = jnp.max(s8, axis=1, keepdims=True)
    e = jnp.exp(s8 - m)
    sm = e / jnp.sum(e, axis=1, keepdims=True)
    e8i = e8.astype(jnp.int32)
    idx_ref[...] = jnp.concatenate(
        [e8i[hh * _BT:(hh + 1) * _BT] for hh in range(_H)], axis=1)
    prob_ref[...] = jnp.concatenate(
        [sm[hh * _BT:(hh + 1) * _BT] for hh in range(_H)], axis=1)


def _routing(h2, wq, k2, interpret=False):
    return pl.pallas_call(
        _routing_body,
        grid=(_T // _BT,),
        in_specs=[
            pl.BlockSpec((_BT, _D), lambda i: (i, 0)),
            pl.BlockSpec(wq.shape, lambda i: (0, 0)),
            pl.BlockSpec(k2.shape, lambda i: (0, 0)),
        ],
        out_specs=[
            pl.BlockSpec((_BT, _SEL), lambda i: (i, 0)),
            pl.BlockSpec((_BT, _SEL), lambda i: (i, 0)),
        ],
        out_shape=[
            jax.ShapeDtypeStruct((_T, _SEL), jnp.int32),
            jax.ShapeDtypeStruct((_T, _SEL), jnp.float32),
        ],
        interpret=interpret,
    )(h2, wq, k2)


def _mlp_body(h_ref, wg_ref, wu_ref, wd_ref, mlp_ref):
    h = h_ref[...].astype(jnp.bfloat16)
    g = jnp.dot(h, wg_ref[...], preferred_element_type=jnp.float32)
    u = jnp.dot(h, wu_ref[...], preferred_element_type=jnp.float32)
    a = ((g / (1.0 + jnp.exp(-g))) * u).astype(jnp.bfloat16)
    mlp_ref[...] = jnp.dot(a, wd_ref[...], preferred_element_type=jnp.float32)


def _mlp(h2, wg, wu, wd, interpret=False):
    return pl.pallas_call(
        _mlp_body,
        grid=(_T // _BT,),
        in_specs=[
            pl.BlockSpec((_BT, _D), lambda i: (i, 0)),
            pl.BlockSpec(wg.shape, lambda i: (0, 0)),
            pl.BlockSpec(wu.shape, lambda i: (0, 0)),
            pl.BlockSpec(wd.shape, lambda i: (0, 0)),
        ],
        out_specs=pl.BlockSpec((_BT, _D), lambda i: (i, 0)),
        out_shape=jax.ShapeDtypeStruct((_T, _D), jnp.float32),
        interpret=interpret,
    )(h2, wg, wu, wd)


def _add_body(a_ref, b_ref, o_ref):
    o_ref[...] = a_ref[...] + b_ref[...]


def _add(a, b, interpret=False):
    return pl.pallas_call(
        _add_body,
        grid=(_T // _BT,),
        in_specs=[
            pl.BlockSpec((_BT, _D), lambda i: (i, 0)),
            pl.BlockSpec((_BT, _D), lambda i: (i, 0)),
        ],
        out_specs=pl.BlockSpec((_BT, _D), lambda i: (i, 0)),
        out_shape=jax.ShapeDtypeStruct((_T, _D), jnp.float32),
        interpret=interpret,
    )(a, b)


def _sc_experts_body(h_hbm, idx_hbm, prob_hbm, down_hbm, up_hbm, out_hbm,
                     idx_all, prob_all, h_v, out_v, dn0, dn1, up_v,
                     sem_h0, sem_h1, sem_d0, sem_d1, sem_u, sem_o0, sem_o1):
    wid = lax.axis_index("s") * _NC + lax.axis_index("c")
    t0 = wid * _TPW
    lane = lax.iota(jnp.int32, 16)
    sos = (sem_o0, sem_o1)

    pltpu.sync_copy(idx_hbm.at[pl.ds(t0, _TPW)], idx_all)
    pltpu.sync_copy(prob_hbm.at[pl.ds(t0, _TPW)], prob_all)

    pltpu.async_copy(down_hbm.at[idx_all.at[0]], dn0, sem_d0)
    pltpu.async_copy(h_hbm.at[t0], h_v.at[0], sem_h0)

    dns = (dn0, dn1)
    sds = (sem_d0, sem_d1)
    shs = (sem_h0, sem_h1)

    def process(i, b):
        t = t0 + i
        dn_cur, dn_nxt = dns[b], dns[1 - b]
        sd_cur, sd_nxt = sds[b], sds[1 - b]
        sh_cur, sh_nxt = shs[b], shs[1 - b]

        pltpu.async_copy(up_hbm.at[idx_all.at[i]], up_v, sem_u)

        @pl.when(i + 1 < _TPW)
        def _():
            pltpu.async_copy(down_hbm.at[idx_all.at[i + 1]], dn_nxt, sd_nxt)
            pltpu.async_copy(h_hbm.at[t + 1], h_v.at[1 - b], sh_nxt)

        pltpu.make_async_copy(down_hbm.at[idx_all.at[i]], dn_cur, sd_cur).wait()
        pltpu.make_async_copy(h_hbm.at[t], h_v.at[b], sh_cur).wait()

        def dot_chunk(cidx, accs):
            hc = h_v[b, pl.ds(cidx * 16, 16)]
            return tuple(accs[j] + hc * dn_cur[j, pl.ds(cidx * 16, 16)]
                         for j in range(_SEL))

        zero = jnp.zeros((16,), jnp.float32)
        accs = lax.fori_loop(0, _D // 16, dot_chunk, (zero,) * _SEL)

        ews = []
        for j in range(_SEL):
            dj = jnp.sum(accs[j])
            pv = prob_all[i, pl.ds((j // 16) * 16, 16)]
            pj = jnp.sum(jnp.where(lane == (j % 16), pv, 0.0))
            v = jnp.full((16,), dj)
            sv = v / (1.0 + jnp.exp(-v))
            sj = jnp.sum(sv) * (1.0 / 16.0)
            ews.append(sj * pj)

        pltpu.make_async_copy(up_hbm.at[idx_all.at[i]], up_v, sem_u).wait()

        @pl.when(i >= 2)
        def _():
            pltpu.make_async_copy(out_v.at[b], out_hbm.at[t - 2],
                                  sos[b]).wait()

        def up_chunk(cidx, c):
            o = ews[0] * up_v[0, pl.ds(cidx * 16, 16)]
            for j in range(1, _SEL):
                o = o + ews[j] * up_v[j, pl.ds(cidx * 16, 16)]
            out_v[b, pl.ds(cidx * 16, 16)] = o
            return c

        lax.fori_loop(0, _D // 16, up_chunk, 0)
        pltpu.async_copy(out_v.at[b], out_hbm.at[t], sos[b])

    def pair(k, carry):
        process(2 * k, 0)
        process(2 * k + 1, 1)
        return carry

    lax.fori_loop(0, _TPW // 2, pair, 0)
    pltpu.make_async_copy(out_v.at[0], out_hbm.at[t0 + _TPW - 2], sos[0]).wait()
    pltpu.make_async_copy(out_v.at[1], out_hbm.at[t0 + _TPW - 1], sos[1]).wait()


def _sc_experts(h2, idx, prob, down, up):
    mesh = plsc.VectorSubcoreMesh(core_axis_name="c", subcore_axis_name="s",
                                  num_cores=_NC, num_subcores=_NS)
    run = pl.kernel(
        _sc_experts_body,
        out_type=jax.ShapeDtypeStruct((_T, _D), jnp.float32),
        mesh=mesh,
        scratch_types=[
            pltpu.VMEM((_TPW, _SEL), jnp.int32),
            pltpu.VMEM((_TPW, _SEL), jnp.float32),
            pltpu.VMEM((2, _D), jnp.float32),
            pltpu.VMEM((2, _D), jnp.float32),
            pltpu.VMEM((_SEL, _D), jnp.float32),
            pltpu.VMEM((_SEL, _D), jnp.float32),
            pltpu.VMEM((_SEL, _D), jnp.float32),
            pltpu.SemaphoreType.DMA,
            pltpu.SemaphoreType.DMA,
            pltpu.SemaphoreType.DMA,
            pltpu.SemaphoreType.DMA,
            pltpu.SemaphoreType.DMA,
            pltpu.SemaphoreType.DMA,
            pltpu.SemaphoreType.DMA,
        ],
        compiler_params=pltpu.CompilerParams(needs_layout_passes=False),
    )
    return run(h2, idx, prob, down, up)


def kernel(hidden_states, Wq, keys_p, down_embed, up_embed, Wg, Wu, Wd):
    b, t, d = hidden_states.shape
    h2 = hidden_states.reshape(t, d)
    k2 = jnp.transpose(keys_p, (2, 0, 3, 1)).reshape(2 * _H * 32, _NK)
    idx, prob = _routing(h2, Wq, k2)
    experts = _sc_experts(h2, idx, prob, down_embed, up_embed)
    mlp = _mlp(h2, Wg.astype(jnp.bfloat16), Wu.astype(jnp.bfloat16),
               Wd.astype(jnp.bfloat16))
    out = _add(mlp, experts)
    return out.reshape(b, t, d)
